# baseline (speedup 1.0000x reference)
import jax
import jax.numpy as jnp
from jax.experimental import pallas as pl


def _sage(x, src, dst, Wl, Wr, b):
    n = x.shape[0]
    msgs = jnp.take(x, src, axis=0)
    agg = jax.ops.segment_sum(msgs, dst, num_segments=n)
    cnt = jax.ops.segment_sum(jnp.ones((src.shape[0],), x.dtype), dst, num_segments=n)
    agg = agg / jnp.maximum(cnt, 1.0)[:, None]
    return agg @ Wl + x @ Wr + b


def kernel(x, edge_index, Wl0, Wr0, b0, Wl1, Wr1, b1, Wl2, Wr2, b2):
    src = edge_index[0]
    dst = edge_index[1]
    h = jax.nn.relu(_sage(x, src, dst, Wl0, Wr0, b0))
    h = jax.nn.relu(_sage(h, src, dst, Wl1, Wr1, b1))
    return _sage(h, src, dst, Wl2, Wr2, b2)


# speedup vs baseline: 3.0520x; 3.0520x over previous
import functools

import jax
import jax.numpy as jnp
from jax import lax
from jax.experimental import pallas as pl
from jax.experimental.pallas import tpu as pltpu
from jax.experimental.pallas import tpu_sc as plsc

N = 10000
E = 160000
D = 256
H = D // 2
NC, NS = 2, 16
CH = 128
NCHUNK = 80
EPT = NCHUNK * CH
E_PAD = EPT * NS
NP = 10240
ZROWS = NP // NS


def _zero_rows(ref, nrows, ncol16):
    def body(i, _):
        for j in range(ncol16):
            ref[i, pl.ds(j * 16, 16)] = jnp.zeros((16,), jnp.float32)
        return 0
    lax.fori_loop(0, nrows, body, 0)


def _fill_ones(ref, nrows, ncol16):
    def body(i, _):
        for j in range(ncol16):
            ref[i, pl.ds(j * 16, 16)] = jnp.ones((16,), jnp.float32)
        return 0
    lax.fori_loop(0, nrows, body, 0)


def _sc_agg_body(h0, h1, srcg, dstg, agg0, agg1, src_v, dst_v, rows_v, acc, sem):
    c = lax.axis_index("c")
    s = lax.axis_index("s")

    _zero_rows(rows_v, CH, H // 16)
    for k in range(ZROWS // CH):
        pltpu.sync_copy(rows_v, acc.at[pl.ds(s * ZROWS + k * CH, CH)])
    plsc.subcore_barrier()

    pltpu.sync_copy(srcg.at[pl.ds(s * NCHUNK, NCHUNK)], src_v)
    pltpu.sync_copy(dstg.at[pl.ds(s * NCHUNK, NCHUNK)], dst_v)

    def run(h_ref):
        def chunk(j, _):
            pltpu.async_copy(h_ref.at[src_v.at[j]], rows_v, sem).wait()
            pltpu.sync_copy(rows_v, acc.at[dst_v.at[j]], add=True)
            return 0
        lax.fori_loop(0, NCHUNK, chunk, 0)

    @pl.when(c == 0)
    def _():
        run(h0)

    @pl.when(c == 1)
    def _():
        run(h1)

    plsc.subcore_barrier()

    r0 = s * ZROWS

    @pl.when(c == 0)
    def _():
        pltpu.sync_copy(acc.at[pl.ds(r0, ZROWS)], agg0.at[pl.ds(r0, ZROWS)])

    @pl.when(c == 1)
    def _():
        pltpu.sync_copy(acc.at[pl.ds(r0, ZROWS)], agg1.at[pl.ds(r0, ZROWS)])


_sc_agg = pl.kernel(
    _sc_agg_body,
    out_type=(
        jax.ShapeDtypeStruct((NP, H), jnp.float32),
        jax.ShapeDtypeStruct((NP, H), jnp.float32),
    ),
    mesh=plsc.VectorSubcoreMesh(core_axis_name="c", subcore_axis_name="s"),
    scratch_types=[
        pltpu.VMEM((NCHUNK, CH), jnp.int32),
        pltpu.VMEM((NCHUNK, CH), jnp.int32),
        pltpu.VMEM((CH, H), jnp.float32),
        pltpu.VMEM_SHARED((NP, H), jnp.float32),
        pltpu.SemaphoreType.DMA,
    ],
)

CNT_CHUNKS = E_PAD // CH // (NC * NS)


def _sc_cnt_body(dstg, cntp, dst_v, ones_v, cnt_acc, sem):
    c = lax.axis_index("c")
    s = lax.axis_index("s")
    wid = c * NS + s

    _zero_rows(ones_v, CH, H // 16)
    for k in range(ZROWS // CH):
        pltpu.sync_copy(ones_v, cnt_acc.at[pl.ds(s * ZROWS + k * CH, CH)])
    _fill_ones(ones_v, CH, H // 16)
    plsc.subcore_barrier()

    pltpu.sync_copy(dstg.at[pl.ds(wid * CNT_CHUNKS, CNT_CHUNKS)], dst_v)

    def chunk(j, _):
        pltpu.sync_copy(ones_v, cnt_acc.at[dst_v.at[j]], add=True)
        return 0
    lax.fori_loop(0, CNT_CHUNKS, chunk, 0)

    plsc.subcore_barrier()
    off = pl.multiple_of(c * NP + s * ZROWS, ZROWS)
    pltpu.sync_copy(cnt_acc.at[pl.ds(s * ZROWS, ZROWS)], cntp.at[pl.ds(off, ZROWS)])


_sc_cnt = pl.kernel(
    _sc_cnt_body,
    out_type=jax.ShapeDtypeStruct((2 * NP, H), jnp.float32),
    mesh=plsc.VectorSubcoreMesh(core_axis_name="c", subcore_axis_name="s"),
    scratch_types=[
        pltpu.VMEM((CNT_CHUNKS, CH), jnp.int32),
        pltpu.VMEM((CH, H), jnp.float32),
        pltpu.VMEM_SHARED((NP, H), jnp.float32),
        pltpu.SemaphoreType.DMA,
    ],
)


def _tc_layer_body(relu, split, a0, a1, x0, x1, ca, cb, wl, wr, bb, *outs):
    deg = jnp.maximum(ca[:, 0:1] + cb[:, 0:1], 1.0)
    agg = jnp.concatenate([a0[...], a1[...]], axis=1) / deg
    xc = jnp.concatenate([x0[...], x1[...]], axis=1)
    out = (jnp.dot(agg, wl[...], preferred_element_type=jnp.float32)
           + jnp.dot(xc, wr[...], preferred_element_type=jnp.float32)
           + bb[...])
    if relu:
        out = jnp.maximum(out, 0.0)
    if split:
        outs[0][...] = out[:, :H]
        outs[1][...] = out[:, H:]
    else:
        outs[0][...] = out


def _make_tc_layer(relu, split, block_m=1024):
    grid = (NP // block_m,)
    row = lambda i: (i, 0)
    fixed = lambda i: (0, 0)
    in_specs = [
        pl.BlockSpec((block_m, H), row),
        pl.BlockSpec((block_m, H), row),
        pl.BlockSpec((block_m, H), row),
        pl.BlockSpec((block_m, H), row),
        pl.BlockSpec((block_m, H), row),
        pl.BlockSpec((block_m, H), lambda i: (i + NP // block_m, 0)),
        pl.BlockSpec((D, D), fixed),
        pl.BlockSpec((D, D), fixed),
        pl.BlockSpec((1, D), fixed),
    ]
    if split:
        out_shape = [jax.ShapeDtypeStruct((NP, H), jnp.float32)] * 2
        out_specs = [pl.BlockSpec((block_m, H), row)] * 2
    else:
        out_shape = [jax.ShapeDtypeStruct((NP, D), jnp.float32)]
        out_specs = [pl.BlockSpec((block_m, D), row)]
    return pl.pallas_call(
        functools.partial(_tc_layer_body, relu, split),
        grid=grid, in_specs=in_specs, out_specs=out_specs, out_shape=out_shape,
    )


_tc_mid = _make_tc_layer(True, True)
_tc_last = _make_tc_layer(False, False)


def kernel(x, edge_index, Wl0, Wr0, b0, Wl1, Wr1, b1, Wl2, Wr2, b2):
    src = edge_index[0].astype(jnp.int32)
    dst = edge_index[1].astype(jnp.int32)
    pad = E_PAD - E
    srcg = jnp.concatenate([src, jnp.zeros((pad,), jnp.int32)]).reshape(-1, CH)
    dstg = jnp.concatenate([dst, jnp.full((pad,), N, jnp.int32)]).reshape(-1, CH)

    xp = jnp.concatenate([x, jnp.zeros((NP - N, D), jnp.float32)])
    h0, h1 = xp[:, :H], xp[:, H:]
    b0r, b1r, b2r = b0.reshape(1, D), b1.reshape(1, D), b2.reshape(1, D)

    cntp = _sc_cnt(dstg)

    agg0, agg1 = _sc_agg(h0, h1, srcg, dstg)
    h0, h1 = _tc_mid(agg0, agg1, h0, h1, cntp, cntp, Wl0, Wr0, b0r)

    agg0, agg1 = _sc_agg(h0, h1, srcg, dstg)
    h0, h1 = _tc_mid(agg0, agg1, h0, h1, cntp, cntp, Wl1, Wr1, b1r)

    agg0, agg1 = _sc_agg(h0, h1, srcg, dstg)
    (out,) = _tc_last(agg0, agg1, h0, h1, cntp, cntp, Wl2, Wr2, b2r)
    return out[:N]


# speedup vs baseline: 3.5140x; 1.1514x over previous
import functools

import jax
import jax.numpy as jnp
from jax import lax
from jax.experimental import pallas as pl
from jax.experimental.pallas import tpu as pltpu
from jax.experimental.pallas import tpu_sc as plsc

N = 10000
E = 160000
D = 256
H = D // 2
NC, NS = 2, 16
CH = 128
NCHUNK = 80
EPT = NCHUNK * CH
E_PAD = EPT * NS
NP = 10240
ZROWS = NP // NS
NB = 2
IB = 16


def _zero_rows(ref, nrows, ncol16):
    def body(i, _):
        for j in range(ncol16):
            ref[i, pl.ds(j * 16, 16)] = jnp.zeros((16,), jnp.float32)
        return 0
    lax.fori_loop(0, nrows, body, 0)


def _fill_ones(ref, nrows, ncol16):
    def body(i, _):
        for j in range(ncol16):
            ref[i, pl.ds(j * 16, 16)] = jnp.ones((16,), jnp.float32)
        return 0
    lax.fori_loop(0, nrows, body, 0)


def _sc_agg_body(h0, h1, srcg, dstg, agg0, agg1, src_v, dst_v, rows_v, acc,
                 semg, sems):
    c = lax.axis_index("c")
    s = lax.axis_index("s")

    _zero_rows(rows_v.at[0], CH, H // 16)
    for k in range(ZROWS // CH):
        pltpu.sync_copy(rows_v.at[0], acc.at[pl.ds(s * ZROWS + k * CH, CH)])
    plsc.subcore_barrier()

    def run(h_ref):
        def block(g, _):
            @pl.when(g > 0)
            def _():
                for b in range(NB):
                    pltpu.make_async_copy(
                        rows_v.at[b], acc.at[dst_v.at[b]], sems).wait()
            pltpu.sync_copy(srcg.at[pl.ds(s * NCHUNK + g * IB, IB)], src_v)
            pltpu.sync_copy(dstg.at[pl.ds(s * NCHUNK + g * IB, IB)], dst_v)

            def group(k, _):
                for b in range(NB):
                    j = k * NB + b

                    @pl.when(k > 0)
                    def _():
                        pltpu.make_async_copy(
                            rows_v.at[b], acc.at[dst_v.at[j]], sems).wait()
                    pltpu.async_copy(h_ref.at[src_v.at[j]], rows_v.at[b], semg)
                for b in range(NB):
                    j = k * NB + b
                    pltpu.make_async_copy(
                        h_ref.at[src_v.at[j]], rows_v.at[b], semg).wait()
                    pltpu.async_copy(rows_v.at[b], acc.at[dst_v.at[j]], sems,
                                     add=True)
                return 0
            lax.fori_loop(0, IB // NB, group, 0)
            return 0
        lax.fori_loop(0, NCHUNK // IB, block, 0)
        for b in range(NB):
            pltpu.make_async_copy(rows_v.at[b], acc.at[dst_v.at[b]], sems).wait()

    @pl.when(c == 0)
    def _():
        run(h0)

    @pl.when(c == 1)
    def _():
        run(h1)

    plsc.subcore_barrier()

    r0 = s * ZROWS

    @pl.when(c == 0)
    def _():
        pltpu.sync_copy(acc.at[pl.ds(r0, ZROWS)], agg0.at[pl.ds(r0, ZROWS)])

    @pl.when(c == 1)
    def _():
        pltpu.sync_copy(acc.at[pl.ds(r0, ZROWS)], agg1.at[pl.ds(r0, ZROWS)])


_sc_agg = pl.kernel(
    _sc_agg_body,
    out_type=(
        jax.ShapeDtypeStruct((NP, H), jnp.float32),
        jax.ShapeDtypeStruct((NP, H), jnp.float32),
    ),
    mesh=plsc.VectorSubcoreMesh(core_axis_name="c", subcore_axis_name="s"),
    scratch_types=[
        pltpu.VMEM((IB, CH), jnp.int32),
        pltpu.VMEM((IB, CH), jnp.int32),
        pltpu.VMEM((NB, CH, H), jnp.float32),
        pltpu.VMEM_SHARED((NP, H), jnp.float32),
        pltpu.SemaphoreType.DMA,
        pltpu.SemaphoreType.DMA,
    ],
)

CNT_CHUNKS = E_PAD // CH // (NC * NS)


def _sc_cnt_body(dstg, cntp, dst_v, ones_v, cnt_acc, sem):
    c = lax.axis_index("c")
    s = lax.axis_index("s")
    wid = c * NS + s

    _zero_rows(ones_v, CH, H // 16)
    for k in range(ZROWS // CH):
        pltpu.sync_copy(ones_v, cnt_acc.at[pl.ds(s * ZROWS + k * CH, CH)])
    _fill_ones(ones_v, CH, H // 16)
    plsc.subcore_barrier()

    pltpu.sync_copy(dstg.at[pl.ds(wid * CNT_CHUNKS, CNT_CHUNKS)], dst_v)

    def chunk(j, _):
        pltpu.sync_copy(ones_v, cnt_acc.at[dst_v.at[j]], add=True)
        return 0
    lax.fori_loop(0, CNT_CHUNKS, chunk, 0)

    plsc.subcore_barrier()
    off = pl.multiple_of(c * NP + s * ZROWS, ZROWS)
    pltpu.sync_copy(cnt_acc.at[pl.ds(s * ZROWS, ZROWS)], cntp.at[pl.ds(off, ZROWS)])


_sc_cnt = pl.kernel(
    _sc_cnt_body,
    out_type=jax.ShapeDtypeStruct((2 * NP, H), jnp.float32),
    mesh=plsc.VectorSubcoreMesh(core_axis_name="c", subcore_axis_name="s"),
    scratch_types=[
        pltpu.VMEM((CNT_CHUNKS, CH), jnp.int32),
        pltpu.VMEM((CH, H), jnp.float32),
        pltpu.VMEM_SHARED((NP, H), jnp.float32),
        pltpu.SemaphoreType.DMA,
    ],
)


def _tc_layer_body(relu, split, a0, a1, x0, x1, ca, cb, wl, wr, bb, *outs):
    deg = jnp.maximum(ca[:, 0:1] + cb[:, 0:1], 1.0)
    agg = jnp.concatenate([a0[...], a1[...]], axis=1) / deg
    xc = jnp.concatenate([x0[...], x1[...]], axis=1)
    out = (jnp.dot(agg, wl[...], preferred_element_type=jnp.float32)
           + jnp.dot(xc, wr[...], preferred_element_type=jnp.float32)
           + bb[...])
    if relu:
        out = jnp.maximum(out, 0.0)
    if split:
        outs[0][...] = out[:, :H]
        outs[1][...] = out[:, H:]
    else:
        outs[0][...] = out


def _make_tc_layer(relu, split, block_m=1024):
    grid = (NP // block_m,)
    row = lambda i: (i, 0)
    fixed = lambda i: (0, 0)
    in_specs = [
        pl.BlockSpec((block_m, H), row),
        pl.BlockSpec((block_m, H), row),
        pl.BlockSpec((block_m, H), row),
        pl.BlockSpec((block_m, H), row),
        pl.BlockSpec((block_m, H), row),
        pl.BlockSpec((block_m, H), lambda i: (i + NP // block_m, 0)),
        pl.BlockSpec((D, D), fixed),
        pl.BlockSpec((D, D), fixed),
        pl.BlockSpec((1, D), fixed),
    ]
    if split:
        out_shape = [jax.ShapeDtypeStruct((NP, H), jnp.float32)] * 2
        out_specs = [pl.BlockSpec((block_m, H), row)] * 2
    else:
        out_shape = [jax.ShapeDtypeStruct((NP, D), jnp.float32)]
        out_specs = [pl.BlockSpec((block_m, D), row)]
    return pl.pallas_call(
        functools.partial(_tc_layer_body, relu, split),
        grid=grid, in_specs=in_specs, out_specs=out_specs, out_shape=out_shape,
    )


_tc_mid = _make_tc_layer(True, True)
_tc_last = _make_tc_layer(False, False)


def kernel(x, edge_index, Wl0, Wr0, b0, Wl1, Wr1, b1, Wl2, Wr2, b2):
    src = edge_index[0].astype(jnp.int32)
    dst = edge_index[1].astype(jnp.int32)
    pad = E_PAD - E
    srcg = jnp.concatenate([src, jnp.zeros((pad,), jnp.int32)]).reshape(-1, CH)
    dstg = jnp.concatenate([dst, jnp.full((pad,), N, jnp.int32)]).reshape(-1, CH)

    xp = jnp.concatenate([x, jnp.zeros((NP - N, D), jnp.float32)])
    h0, h1 = xp[:, :H], xp[:, H:]
    b0r, b1r, b2r = b0.reshape(1, D), b1.reshape(1, D), b2.reshape(1, D)

    cntp = _sc_cnt(dstg)

    agg0, agg1 = _sc_agg(h0, h1, srcg, dstg)
    h0, h1 = _tc_mid(agg0, agg1, h0, h1, cntp, cntp, Wl0, Wr0, b0r)

    agg0, agg1 = _sc_agg(h0, h1, srcg, dstg)
    h0, h1 = _tc_mid(agg0, agg1, h0, h1, cntp, cntp, Wl1, Wr1, b1r)

    agg0, agg1 = _sc_agg(h0, h1, srcg, dstg)
    (out,) = _tc_last(agg0, agg1, h0, h1, cntp, cntp, Wl2, Wr2, b2r)
    return out[:N]


# speedup vs baseline: 3.5489x; 1.0099x over previous
import functools

import jax
import jax.numpy as jnp
from jax import lax
from jax.experimental import pallas as pl
from jax.experimental.pallas import tpu as pltpu
from jax.experimental.pallas import tpu_sc as plsc

N = 10000
E = 160000
D = 256
H = D // 2
NC, NS = 2, 16
CH = 128
NCHUNK = 80
EPT = NCHUNK * CH
E_PAD = EPT * NS
NP = 10240
ZROWS = NP // NS
NB = 2
IB = 40


def _zero_rows(ref, nrows, ncol16):
    def body(i, _):
        for j in range(ncol16):
            ref[i, pl.ds(j * 16, 16)] = jnp.zeros((16,), jnp.float32)
        return 0
    lax.fori_loop(0, nrows, body, 0)


def _fill_ones(ref, nrows, ncol16):
    def body(i, _):
        for j in range(ncol16):
            ref[i, pl.ds(j * 16, 16)] = jnp.ones((16,), jnp.float32)
        return 0
    lax.fori_loop(0, nrows, body, 0)


def _sc_agg_body(h0, h1, srcg, dstg, agg0, agg1, src_v, dst_v, rows_v, acc,
                 semg, sems):
    c = lax.axis_index("c")
    s = lax.axis_index("s")

    _zero_rows(rows_v.at[0], CH, H // 16)
    for k in range(ZROWS // CH):
        pltpu.sync_copy(rows_v.at[0], acc.at[pl.ds(s * ZROWS + k * CH, CH)])
    plsc.subcore_barrier()

    def run(h_ref):
        def block(g, _):
            @pl.when(g > 0)
            def _():
                for b in range(NB):
                    pltpu.make_async_copy(
                        rows_v.at[b], acc.at[dst_v.at[b]], sems).wait()
            pltpu.sync_copy(srcg.at[pl.ds(s * NCHUNK + g * IB, IB)], src_v)
            pltpu.sync_copy(dstg.at[pl.ds(s * NCHUNK + g * IB, IB)], dst_v)

            def group(k, _):
                for b in range(NB):
                    j = k * NB + b

                    @pl.when(k > 0)
                    def _():
                        pltpu.make_async_copy(
                            rows_v.at[b], acc.at[dst_v.at[j]], sems).wait()
                    pltpu.async_copy(h_ref.at[src_v.at[j]], rows_v.at[b], semg)
                for b in range(NB):
                    j = k * NB + b
                    pltpu.make_async_copy(
                        h_ref.at[src_v.at[j]], rows_v.at[b], semg).wait()
                    pltpu.async_copy(rows_v.at[b], acc.at[dst_v.at[j]], sems,
                                     add=True)
                return 0
            lax.fori_loop(0, IB // NB, group, 0)
            return 0
        lax.fori_loop(0, NCHUNK // IB, block, 0)
        for b in range(NB):
            pltpu.make_async_copy(rows_v.at[b], acc.at[dst_v.at[b]], sems).wait()

    @pl.when(c == 0)
    def _():
        run(h0)

    @pl.when(c == 1)
    def _():
        run(h1)

    plsc.subcore_barrier()

    r0 = s * ZROWS

    @pl.when(c == 0)
    def _():
        pltpu.sync_copy(acc.at[pl.ds(r0, ZROWS)], agg0.at[pl.ds(r0, ZROWS)])

    @pl.when(c == 1)
    def _():
        pltpu.sync_copy(acc.at[pl.ds(r0, ZROWS)], agg1.at[pl.ds(r0, ZROWS)])


_sc_agg = pl.kernel(
    _sc_agg_body,
    out_type=(
        jax.ShapeDtypeStruct((NP, H), jnp.float32),
        jax.ShapeDtypeStruct((NP, H), jnp.float32),
    ),
    mesh=plsc.VectorSubcoreMesh(core_axis_name="c", subcore_axis_name="s"),
    scratch_types=[
        pltpu.VMEM((IB, CH), jnp.int32),
        pltpu.VMEM((IB, CH), jnp.int32),
        pltpu.VMEM((NB, CH, H), jnp.float32),
        pltpu.VMEM_SHARED((NP, H), jnp.float32),
        pltpu.SemaphoreType.DMA,
        pltpu.SemaphoreType.DMA,
    ],
)

CNT_CHUNKS = E_PAD // CH // (NC * NS)


def _sc_cnt_body(dstg, cntp, dst_v, ones_v, cnt_acc, sem):
    c = lax.axis_index("c")
    s = lax.axis_index("s")
    wid = c * NS + s

    _zero_rows(ones_v, CH, H // 16)
    for k in range(ZROWS // CH):
        pltpu.sync_copy(ones_v, cnt_acc.at[pl.ds(s * ZROWS + k * CH, CH)])
    _fill_ones(ones_v, CH, H // 16)
    plsc.subcore_barrier()

    pltpu.sync_copy(dstg.at[pl.ds(wid * CNT_CHUNKS, CNT_CHUNKS)], dst_v)

    def chunk(j, _):
        pltpu.sync_copy(ones_v, cnt_acc.at[dst_v.at[j]], add=True)
        return 0
    lax.fori_loop(0, CNT_CHUNKS, chunk, 0)

    plsc.subcore_barrier()
    off = pl.multiple_of(c * NP + s * ZROWS, ZROWS)
    pltpu.sync_copy(cnt_acc.at[pl.ds(s * ZROWS, ZROWS)], cntp.at[pl.ds(off, ZROWS)])


_sc_cnt = pl.kernel(
    _sc_cnt_body,
    out_type=jax.ShapeDtypeStruct((2 * NP, H), jnp.float32),
    mesh=plsc.VectorSubcoreMesh(core_axis_name="c", subcore_axis_name="s"),
    scratch_types=[
        pltpu.VMEM((CNT_CHUNKS, CH), jnp.int32),
        pltpu.VMEM((CH, H), jnp.float32),
        pltpu.VMEM_SHARED((NP, H), jnp.float32),
        pltpu.SemaphoreType.DMA,
    ],
)


def _tc_layer_body(relu, split, a0, a1, x0, x1, ca, cb, wl, wr, bb, *outs):
    deg = jnp.maximum(ca[:, 0:1] + cb[:, 0:1], 1.0)
    agg = jnp.concatenate([a0[...], a1[...]], axis=1) / deg
    xc = jnp.concatenate([x0[...], x1[...]], axis=1)
    out = (jnp.dot(agg, wl[...], preferred_element_type=jnp.float32)
           + jnp.dot(xc, wr[...], preferred_element_type=jnp.float32)
           + bb[...])
    if relu:
        out = jnp.maximum(out, 0.0)
    if split:
        outs[0][...] = out[:, :H]
        outs[1][...] = out[:, H:]
    else:
        outs[0][...] = out


def _make_tc_layer(relu, split, block_m=1024):
    grid = (NP // block_m,)
    row = lambda i: (i, 0)
    fixed = lambda i: (0, 0)
    in_specs = [
        pl.BlockSpec((block_m, H), row),
        pl.BlockSpec((block_m, H), row),
        pl.BlockSpec((block_m, H), row),
        pl.BlockSpec((block_m, H), row),
        pl.BlockSpec((block_m, H), row),
        pl.BlockSpec((block_m, H), lambda i: (i + NP // block_m, 0)),
        pl.BlockSpec((D, D), fixed),
        pl.BlockSpec((D, D), fixed),
        pl.BlockSpec((1, D), fixed),
    ]
    if split:
        out_shape = [jax.ShapeDtypeStruct((NP, H), jnp.float32)] * 2
        out_specs = [pl.BlockSpec((block_m, H), row)] * 2
    else:
        out_shape = [jax.ShapeDtypeStruct((NP, D), jnp.float32)]
        out_specs = [pl.BlockSpec((block_m, D), row)]
    return pl.pallas_call(
        functools.partial(_tc_layer_body, relu, split),
        grid=grid, in_specs=in_specs, out_specs=out_specs, out_shape=out_shape,
    )


_tc_mid = _make_tc_layer(True, True)
_tc_last = _make_tc_layer(False, False)


def kernel(x, edge_index, Wl0, Wr0, b0, Wl1, Wr1, b1, Wl2, Wr2, b2):
    src = edge_index[0].astype(jnp.int32)
    dst = edge_index[1].astype(jnp.int32)
    pad = E_PAD - E
    srcg = jnp.concatenate([src, jnp.zeros((pad,), jnp.int32)]).reshape(-1, CH)
    dstg = jnp.concatenate([dst, jnp.full((pad,), N, jnp.int32)]).reshape(-1, CH)

    xp = jnp.concatenate([x, jnp.zeros((NP - N, D), jnp.float32)])
    h0, h1 = xp[:, :H], xp[:, H:]
    b0r, b1r, b2r = b0.reshape(1, D), b1.reshape(1, D), b2.reshape(1, D)

    cntp = _sc_cnt(dstg)

    agg0, agg1 = _sc_agg(h0, h1, srcg, dstg)
    h0, h1 = _tc_mid(agg0, agg1, h0, h1, cntp, cntp, Wl0, Wr0, b0r)

    agg0, agg1 = _sc_agg(h0, h1, srcg, dstg)
    h0, h1 = _tc_mid(agg0, agg1, h0, h1, cntp, cntp, Wl1, Wr1, b1r)

    agg0, agg1 = _sc_agg(h0, h1, srcg, dstg)
    (out,) = _tc_last(agg0, agg1, h0, h1, cntp, cntp, Wl2, Wr2, b2r)
    return out[:N]


# speedup vs baseline: 3.5725x; 1.0066x over previous
import functools

import jax
import jax.numpy as jnp
from jax import lax
from jax.experimental import pallas as pl
from jax.experimental.pallas import tpu as pltpu
from jax.experimental.pallas import tpu_sc as plsc

N = 10000
E = 160000
D = 256
H = D // 2
NC, NS = 2, 16
CH = 128
NCHUNK = 80
EPT = NCHUNK * CH
E_PAD = EPT * NS
NP = 10240
ZROWS = NP // NS
NB = 2
IB = 40


def _zero_rows(ref, nrows, ncol16):
    def body(i, _):
        for j in range(ncol16):
            ref[i, pl.ds(j * 16, 16)] = jnp.zeros((16,), jnp.float32)
        return 0
    lax.fori_loop(0, nrows, body, 0)


def _fill_ones(ref, nrows, ncol16):
    def body(i, _):
        for j in range(ncol16):
            ref[i, pl.ds(j * 16, 16)] = jnp.ones((16,), jnp.float32)
        return 0
    lax.fori_loop(0, nrows, body, 0)


def _sc_agg_body(h0, h1, srcg, dstg, agg0, agg1, src_v, dst_v, rows_v, acc,
                 semg, sems):
    c = lax.axis_index("c")
    s = lax.axis_index("s")

    _zero_rows(rows_v.at[0], CH, H // 16)
    for k in range(ZROWS // CH):
        pltpu.sync_copy(rows_v.at[0], acc.at[pl.ds(s * ZROWS + k * CH, CH)])
    plsc.subcore_barrier()

    def run(h_ref):
        def block(g, _):
            @pl.when(g > 0)
            def _():
                for b in range(NB):
                    pltpu.make_async_copy(
                        rows_v.at[b], acc.at[dst_v.at[b]], sems).wait()
            pltpu.sync_copy(srcg.at[pl.ds(s * NCHUNK + g * IB, IB)], src_v)
            pltpu.sync_copy(dstg.at[pl.ds(s * NCHUNK + g * IB, IB)], dst_v)

            def group(k, _):
                for b in range(NB):
                    j = k * NB + b

                    @pl.when(k > 0)
                    def _():
                        pltpu.make_async_copy(
                            rows_v.at[b], acc.at[dst_v.at[j]], sems).wait()
                    pltpu.async_copy(h_ref.at[src_v.at[j]], rows_v.at[b], semg)
                for b in range(NB):
                    j = k * NB + b
                    pltpu.make_async_copy(
                        h_ref.at[src_v.at[j]], rows_v.at[b], semg).wait()
                    pltpu.async_copy(rows_v.at[b], acc.at[dst_v.at[j]], sems,
                                     add=True)
                return 0
            lax.fori_loop(0, IB // NB, group, 0)
            return 0
        lax.fori_loop(0, NCHUNK // IB, block, 0)
        for b in range(NB):
            pltpu.make_async_copy(rows_v.at[b], acc.at[dst_v.at[b]], sems).wait()

    @pl.when(c == 0)
    def _():
        run(h0)

    @pl.when(c == 1)
    def _():
        run(h1)

    plsc.subcore_barrier()

    r0 = s * ZROWS

    @pl.when(c == 0)
    def _():
        pltpu.sync_copy(acc.at[pl.ds(r0, ZROWS)], agg0.at[pl.ds(r0, ZROWS)])

    @pl.when(c == 1)
    def _():
        pltpu.sync_copy(acc.at[pl.ds(r0, ZROWS)], agg1.at[pl.ds(r0, ZROWS)])


_sc_agg = pl.kernel(
    _sc_agg_body,
    out_type=(
        jax.ShapeDtypeStruct((NP, H), jnp.float32),
        jax.ShapeDtypeStruct((NP, H), jnp.float32),
    ),
    mesh=plsc.VectorSubcoreMesh(core_axis_name="c", subcore_axis_name="s"),
    scratch_types=[
        pltpu.VMEM((IB, CH), jnp.int32),
        pltpu.VMEM((IB, CH), jnp.int32),
        pltpu.VMEM((NB, CH, H), jnp.float32),
        pltpu.VMEM_SHARED((NP, H), jnp.float32),
        pltpu.SemaphoreType.DMA,
        pltpu.SemaphoreType.DMA,
    ],
)

CNT_CHUNKS = E_PAD // CH // (NC * NS)


def _sc_cnt_body(dstg, cntp, dst_v, ones_v, cnt_acc, sem):
    c = lax.axis_index("c")
    s = lax.axis_index("s")
    wid = c * NS + s

    _zero_rows(ones_v, CH, H // 16)
    for k in range(ZROWS // CH):
        pltpu.sync_copy(ones_v, cnt_acc.at[pl.ds(s * ZROWS + k * CH, CH)])
    _fill_ones(ones_v, CH, H // 16)
    plsc.subcore_barrier()

    pltpu.sync_copy(dstg.at[pl.ds(wid * CNT_CHUNKS, CNT_CHUNKS)], dst_v)

    def chunk(j, _):
        pltpu.sync_copy(ones_v, cnt_acc.at[dst_v.at[j]], add=True)
        return 0
    lax.fori_loop(0, CNT_CHUNKS, chunk, 0)

    plsc.subcore_barrier()
    off = pl.multiple_of(c * NP + s * ZROWS, ZROWS)
    pltpu.sync_copy(cnt_acc.at[pl.ds(s * ZROWS, ZROWS)], cntp.at[pl.ds(off, ZROWS)])


_sc_cnt = pl.kernel(
    _sc_cnt_body,
    out_type=jax.ShapeDtypeStruct((2 * NP, H), jnp.float32),
    mesh=plsc.VectorSubcoreMesh(core_axis_name="c", subcore_axis_name="s"),
    scratch_types=[
        pltpu.VMEM((CNT_CHUNKS, CH), jnp.int32),
        pltpu.VMEM((CH, H), jnp.float32),
        pltpu.VMEM_SHARED((NP, H), jnp.float32),
        pltpu.SemaphoreType.DMA,
    ],
)


def _tc_layer_body(relu, split, a0, a1, x0, x1, ca, cb, wl, wr, bb, *outs):
    deg = jnp.maximum(ca[:, 0:1] + cb[:, 0:1], 1.0)
    agg = jnp.concatenate([a0[...], a1[...]], axis=1) / deg
    xc = jnp.concatenate([x0[...], x1[...]], axis=1)
    out = (jnp.dot(agg, wl[...], preferred_element_type=jnp.float32)
           + jnp.dot(xc, wr[...], preferred_element_type=jnp.float32)
           + bb[...])
    if relu:
        out = jnp.maximum(out, 0.0)
    if split:
        outs[0][...] = out[:, :H]
        outs[1][...] = out[:, H:]
    else:
        outs[0][...] = out


def _make_tc_layer(relu, split, block_m=2048):
    grid = (NP // block_m,)
    row = lambda i: (i, 0)
    fixed = lambda i: (0, 0)
    in_specs = [
        pl.BlockSpec((block_m, H), row),
        pl.BlockSpec((block_m, H), row),
        pl.BlockSpec((block_m, H), row),
        pl.BlockSpec((block_m, H), row),
        pl.BlockSpec((block_m, H), row),
        pl.BlockSpec((block_m, H), lambda i: (i + NP // block_m, 0)),
        pl.BlockSpec((D, D), fixed),
        pl.BlockSpec((D, D), fixed),
        pl.BlockSpec((1, D), fixed),
    ]
    if split:
        out_shape = [jax.ShapeDtypeStruct((NP, H), jnp.float32)] * 2
        out_specs = [pl.BlockSpec((block_m, H), row)] * 2
    else:
        out_shape = [jax.ShapeDtypeStruct((NP, D), jnp.float32)]
        out_specs = [pl.BlockSpec((block_m, D), row)]
    return pl.pallas_call(
        functools.partial(_tc_layer_body, relu, split),
        grid=grid, in_specs=in_specs, out_specs=out_specs, out_shape=out_shape,
    )


_tc_mid = _make_tc_layer(True, True)
_tc_last = _make_tc_layer(False, False)


def kernel(x, edge_index, Wl0, Wr0, b0, Wl1, Wr1, b1, Wl2, Wr2, b2):
    src = edge_index[0].astype(jnp.int32)
    dst = edge_index[1].astype(jnp.int32)
    pad = E_PAD - E
    srcg = jnp.concatenate([src, jnp.zeros((pad,), jnp.int32)]).reshape(-1, CH)
    dstg = jnp.concatenate([dst, jnp.full((pad,), N, jnp.int32)]).reshape(-1, CH)

    xp = jnp.concatenate([x, jnp.zeros((NP - N, D), jnp.float32)])
    h0, h1 = xp[:, :H], xp[:, H:]
    b0r, b1r, b2r = b0.reshape(1, D), b1.reshape(1, D), b2.reshape(1, D)

    cntp = _sc_cnt(dstg)

    agg0, agg1 = _sc_agg(h0, h1, srcg, dstg)
    h0, h1 = _tc_mid(agg0, agg1, h0, h1, cntp, cntp, Wl0, Wr0, b0r)

    agg0, agg1 = _sc_agg(h0, h1, srcg, dstg)
    h0, h1 = _tc_mid(agg0, agg1, h0, h1, cntp, cntp, Wl1, Wr1, b1r)

    agg0, agg1 = _sc_agg(h0, h1, srcg, dstg)
    (out,) = _tc_last(agg0, agg1, h0, h1, cntp, cntp, Wl2, Wr2, b2r)
    return out[:N]
